# baseline (speedup 1.0000x reference)
import functools

import jax
import jax.numpy as jnp
from jax.experimental import pallas as pl
from jax.experimental.pallas import tpu as pltpu

_NE = 64
_TK = 32
_CF = 1.25
_EPS = 1e-06
_TCHUNK = 256


def _gate_kernel(x1_ref, x2_ref, wt_ref, b_ref, o_ref, *, capacity):
    B, S, H = x1_ref.shape
    E = _NE
    T = B * S
    x1 = x1_ref[...].reshape(T, H)
    x2 = x2_ref[...].reshape(T, H)
    logits = jnp.dot(x1, wt_ref[:H], preferred_element_type=jnp.float32)
    logits = logits + jnp.dot(x2, wt_ref[H:], preferred_element_type=jnp.float32)
    logits = logits + b_ref[...]
    lt = logits.T
    m = jnp.max(lt, axis=0, keepdims=True)
    ex = jnp.exp(lt - m)
    g = ex / jnp.sum(ex, axis=0, keepdims=True)
    eidx = jax.lax.broadcasted_iota(jnp.int32, (E, 1), 0)
    masked_parts = []
    for t0 in range(0, T, _TCHUNK):
        gc = g[:, t0:t0 + _TCHUNK]
        rank = jnp.zeros(gc.shape, jnp.int32)
        for j in range(E):
            gj = gc[j:j + 1, :]
            ge = jnp.where(gj >= gc, 1, 0)
            gt = jnp.where(gj > gc, 1, 0)
            rank = rank + jnp.where(eidx > j, ge, gt)
        masked_parts.append(jnp.where(rank < _TK, gc, 0.0))
    masked = jnp.concatenate(masked_parts, axis=1)
    denom = _EPS + sum(masked[:, bb * S:(bb + 1) * S] for bb in range(B))
    scale = capacity / denom
    out_t = jnp.concatenate(
        [masked[:, bb * S:(bb + 1) * S] * scale for bb in range(B)], axis=1)
    o_ref[...] = out_t.T.reshape(B, S, E)


def kernel(X, W, b):
    B, S, D = X.shape
    capacity = int(_CF * B)
    Wt = W.T
    b2 = b.reshape(1, _NE)
    SB = 256
    H = D // 2
    grid = (S // SB,)
    return pl.pallas_call(
        functools.partial(_gate_kernel, capacity=float(capacity)),
        grid=grid,
        in_specs=[
            pl.BlockSpec((B, SB, H), lambda i: (0, i, 0)),
            pl.BlockSpec((B, SB, H), lambda i: (0, i, 1)),
            pl.BlockSpec((D, _NE), lambda i: (0, 0)),
            pl.BlockSpec((1, _NE), lambda i: (0, 0)),
        ],
        out_specs=pl.BlockSpec((B, SB, _NE), lambda i: (0, i, 0)),
        out_shape=jax.ShapeDtypeStruct((B, S, _NE), jnp.float32),
        compiler_params=pltpu.CompilerParams(
            dimension_semantics=("parallel",),
        ),
    )(X, X, Wt, b2)


# speedup vs baseline: 1.0059x; 1.0059x over previous
import functools

import jax
import jax.numpy as jnp
from jax.experimental import pallas as pl
from jax.experimental.pallas import tpu as pltpu

_NE = 64
_TK = 32
_CF = 1.25
_EPS = 1e-06
_TCHUNK = 256


def _gate_kernel(x1_ref, x2_ref, wt_ref, b_ref, o_ref, *, capacity):
    B, SH, D = x1_ref.shape
    E = _NE
    T = 2 * B * SH
    x1 = x1_ref[...].reshape(B * SH, D)
    x2 = x2_ref[...].reshape(B * SH, D)
    wt = wt_ref[...]
    bb_ = b_ref[...]
    l1 = jnp.dot(x1, wt, preferred_element_type=jnp.float32) + bb_
    l2 = jnp.dot(x2, wt, preferred_element_type=jnp.float32) + bb_
    logits = jnp.concatenate([l1, l2], axis=0)
    lt = logits.T
    m = jnp.max(lt, axis=0, keepdims=True)
    ex = jnp.exp(lt - m)
    g = ex / jnp.sum(ex, axis=0, keepdims=True)
    eidx = jax.lax.broadcasted_iota(jnp.int32, (E, 1), 0)
    masked_parts = []
    for t0 in range(0, T, _TCHUNK):
        gc = g[:, t0:t0 + _TCHUNK]
        rank = jnp.zeros(gc.shape, jnp.int32)
        for j in range(E):
            gj = gc[j:j + 1, :]
            ge = jnp.where(gj >= gc, 1, 0)
            gt = jnp.where(gj > gc, 1, 0)
            rank = rank + jnp.where(eidx > j, ge, gt)
        masked_parts.append(jnp.where(rank < _TK, gc, 0.0))
    masked = jnp.concatenate(masked_parts, axis=1)
    halves = []
    for h in range(2):
        mh = masked[:, h * B * SH:(h + 1) * B * SH]
        denom = _EPS + sum(mh[:, bb * SH:(bb + 1) * SH] for bb in range(B))
        scale = capacity / denom
        halves.append(jnp.concatenate(
            [mh[:, bb * SH:(bb + 1) * SH] * scale for bb in range(B)], axis=1))
    o1 = halves[0].T.reshape(B, SH, E)
    o2 = halves[1].T.reshape(B, SH, E)
    o_ref[:, :SH, :] = o1
    o_ref[:, SH:, :] = o2


def kernel(X, W, b):
    B, S, D = X.shape
    capacity = int(_CF * B)
    Wt = W.T
    b2 = b.reshape(1, _NE)
    SB = 256
    SH = SB // 2
    grid = (S // SB,)
    return pl.pallas_call(
        functools.partial(_gate_kernel, capacity=float(capacity)),
        grid=grid,
        in_specs=[
            pl.BlockSpec((B, SH, D), lambda i: (0, 2 * i, 0)),
            pl.BlockSpec((B, SH, D), lambda i: (0, 2 * i + 1, 0)),
            pl.BlockSpec((D, _NE), lambda i: (0, 0)),
            pl.BlockSpec((1, _NE), lambda i: (0, 0)),
        ],
        out_specs=pl.BlockSpec((B, SB, _NE), lambda i: (0, i, 0)),
        out_shape=jax.ShapeDtypeStruct((B, S, _NE), jnp.float32),
        compiler_params=pltpu.CompilerParams(
            dimension_semantics=("parallel",),
        ),
    )(X, X, Wt, b2)
